# Initial kernel scaffold; baseline (speedup 1.0000x reference)
import jax
import jax.numpy as jnp
from jax.experimental import pallas as pl


def kernel(input, linear_table, bias, emb_table, W1, b1, g1, be1, W2, b2, g2, be2, W3, b3):
    raise NotImplementedError("write your pallas kernel here")


# speedup vs baseline: 1.1439x; 1.1439x over previous
import functools

import jax
import jax.numpy as jnp
from jax import lax
from jax.experimental import pallas as pl
from jax.experimental.pallas import tpu as pltpu
from jax.experimental.pallas import tpu_sc as plsc

V = 1000000
F = 26
D = 32
B = 4096
BF = B * F

_NC = 2
_NS = 16
_NW = _NC * _NS
_BPW = BF // _NW


_RB = 4 * D


def _sc_gather_body(idx_hbm, emb_tab, lin_tab, emb_out, lin_out,
                    idx_v, rows_v, lin_v, sem_e, sem_l):
    wid = lax.axis_index("s") * _NC + lax.axis_index("c")
    base = wid * _BPW
    pltpu.sync_copy(idx_hbm.at[pl.ds(base, _BPW)], idx_v)
    cp_e = pltpu.async_copy(emb_tab.at[idx_v], rows_v, sem_e)
    cp_l = pltpu.async_copy(lin_tab.at[idx_v], lin_v, sem_l)
    cp_e.wait()
    cp_l.wait()
    pltpu.sync_copy(rows_v, emb_out.at[pl.ds(base, _BPW)])
    pltpu.sync_copy(lin_v, lin_out.at[pl.ds(base, _BPW)])


_sc_gather = functools.partial(
    pl.kernel,
    out_type=[
        jax.ShapeDtypeStruct((BF, _RB), jnp.int8),
        jax.ShapeDtypeStruct((BF,), jnp.float32),
    ],
    mesh=plsc.VectorSubcoreMesh(core_axis_name="c", subcore_axis_name="s"),
    scratch_types=[
        pltpu.VMEM((_BPW,), jnp.int32),
        pltpu.VMEM((_BPW, _RB), jnp.int8),
        pltpu.VMEM((_BPW,), jnp.float32),
        pltpu.SemaphoreType.DMA,
        pltpu.SemaphoreType.DMA,
    ],
    compiler_params=pltpu.CompilerParams(use_tc_tiling_on_sc=False),
)(_sc_gather_body)


def _mlp_body(emb_ref, lin_ref, bias_ref, w1_ref, b1_ref, g1_ref, be1_ref,
              w2_ref, b2_ref, g2_ref, be2_ref, w3_ref, b3_ref, out_ref):
    eps = 1e-5
    h = emb_ref[...]
    h = jnp.dot(h, w1_ref[...], preferred_element_type=jnp.float32) + b1_ref[...]
    mu = jnp.mean(h, axis=0, keepdims=True)
    var = jnp.mean((h - mu) ** 2, axis=0, keepdims=True)
    h = g1_ref[...] * (h - mu) * lax.rsqrt(var + eps) + be1_ref[...]
    h = jnp.maximum(h, 0.0)
    h = jnp.dot(h, w2_ref[...], preferred_element_type=jnp.float32) + b2_ref[...]
    mu = jnp.mean(h, axis=0, keepdims=True)
    var = jnp.mean((h - mu) ** 2, axis=0, keepdims=True)
    h = g2_ref[...] * (h - mu) * lax.rsqrt(var + eps) + be2_ref[...]
    h = jnp.maximum(h, 0.0)
    deep = jnp.dot(h, w3_ref[...], preferred_element_type=jnp.float32) + b3_ref[...]
    wide = jnp.sum(lin_ref[...], axis=1, keepdims=True)
    out_ref[...] = jax.nn.sigmoid(bias_ref[...] + wide + deep)


_mlp = pl.pallas_call(
    _mlp_body,
    out_shape=jax.ShapeDtypeStruct((B, 1), jnp.float32),
)


def kernel(input, linear_table, bias, emb_table, W1, b1, g1, be1,
           W2, b2, g2, be2, W3, b3):
    idx = input.reshape(BF)
    emb_b = lax.bitcast_convert_type(emb_table, jnp.int8).reshape(V, _RB)
    emb_flat8, lin_flat = _sc_gather(idx, emb_b, linear_table.reshape(V))
    h = lax.bitcast_convert_type(
        emb_flat8.reshape(BF, D, 4), jnp.float32).reshape(B, F * D)
    lin2 = lin_flat.reshape(B, F)
    return _mlp(h, lin2, bias.reshape(1, 1),
                W1, b1.reshape(1, D), g1.reshape(1, D), be1.reshape(1, D),
                W2, b2.reshape(1, D), g2.reshape(1, D), be2.reshape(1, D),
                W3, b3.reshape(1, 1))


# speedup vs baseline: 5.0895x; 4.4494x over previous
import functools

import jax
import jax.numpy as jnp
from jax import lax
from jax.experimental import pallas as pl
from jax.experimental.pallas import tpu as pltpu
from jax.experimental.pallas import tpu_sc as plsc

V = 1000000
F = 26
D = 32
B = 4096
BF = B * F

_NC = 2
_NS = 16
_NW = _NC * _NS
_BPW = BF // _NW
_CH = 416
_NCHUNK = _BPW // _CH


def _sc_gather_body(idx_hbm, emb_tab, lin_tab, emb_out, lin_out,
                    idx_v, rows0, rows1, lin_v, sem0, sem1, sem_l):
    wid = lax.axis_index("s") * _NC + lax.axis_index("c")
    base = wid * _BPW
    pltpu.sync_copy(idx_hbm.at[pl.ds(base, _BPW)], idx_v)
    cp_l = pltpu.async_copy(lin_tab.at[idx_v], lin_v, sem_l)
    bufs = (rows0, rows1)
    sems = (sem0, sem1)
    cps = [None, None]
    cps[0] = pltpu.async_copy(
        emb_tab.at[idx_v.at[pl.ds(0, _CH)]], rows0, sem0)
    for j in range(_NCHUNK):
        cur = j % 2
        if j + 1 < _NCHUNK:
            nxt = (j + 1) % 2
            cps[nxt] = pltpu.async_copy(
                emb_tab.at[idx_v.at[pl.ds((j + 1) * _CH, _CH)]],
                bufs[nxt], sems[nxt])
        cps[cur].wait()
        pltpu.sync_copy(bufs[cur], emb_out.at[pl.ds(base + j * _CH, _CH)])
    cp_l.wait()
    pltpu.sync_copy(lin_v, lin_out.at[pl.ds(base, _BPW)])


_sc_gather = functools.partial(
    pl.kernel,
    out_type=[
        jax.ShapeDtypeStruct((BF, D), jnp.float32),
        jax.ShapeDtypeStruct((BF,), jnp.float32),
    ],
    mesh=plsc.VectorSubcoreMesh(core_axis_name="c", subcore_axis_name="s"),
    scratch_types=[
        pltpu.VMEM((_BPW,), jnp.int32),
        pltpu.VMEM((_CH, D), jnp.float32),
        pltpu.VMEM((_CH, D), jnp.float32),
        pltpu.VMEM((_BPW,), jnp.float32),
        pltpu.SemaphoreType.DMA,
        pltpu.SemaphoreType.DMA,
        pltpu.SemaphoreType.DMA,
    ],
    compiler_params=pltpu.CompilerParams(use_tc_tiling_on_sc=False),
)(_sc_gather_body)


def _mlp_body(emb_ref, lin_ref, bias_ref, w1_ref, b1_ref, g1_ref, be1_ref,
              w2_ref, b2_ref, g2_ref, be2_ref, w3_ref, b3_ref, out_ref):
    eps = 1e-5
    h = emb_ref[...]
    h = jnp.dot(h, w1_ref[...], preferred_element_type=jnp.float32) + b1_ref[...]
    mu = jnp.mean(h, axis=0, keepdims=True)
    var = jnp.mean((h - mu) ** 2, axis=0, keepdims=True)
    h = g1_ref[...] * (h - mu) * lax.rsqrt(var + eps) + be1_ref[...]
    h = jnp.maximum(h, 0.0)
    h = jnp.dot(h, w2_ref[...], preferred_element_type=jnp.float32) + b2_ref[...]
    mu = jnp.mean(h, axis=0, keepdims=True)
    var = jnp.mean((h - mu) ** 2, axis=0, keepdims=True)
    h = g2_ref[...] * (h - mu) * lax.rsqrt(var + eps) + be2_ref[...]
    h = jnp.maximum(h, 0.0)
    deep = jnp.dot(h, w3_ref[...], preferred_element_type=jnp.float32) + b3_ref[...]
    wide = jnp.sum(lin_ref[...], axis=1, keepdims=True)
    out_ref[...] = jax.nn.sigmoid(bias_ref[...] + wide + deep)


_mlp = pl.pallas_call(
    _mlp_body,
    out_shape=jax.ShapeDtypeStruct((B, 1), jnp.float32),
)


def kernel(input, linear_table, bias, emb_table, W1, b1, g1, be1,
           W2, b2, g2, be2, W3, b3):
    idx = input.reshape(BF)
    emb_flat, lin_flat = _sc_gather(idx, emb_table, linear_table.reshape(V))
    h = emb_flat.reshape(B, F * D)
    lin2 = lin_flat.reshape(B, F)
    return _mlp(h, lin2, bias.reshape(1, 1),
                W1, b1.reshape(1, D), g1.reshape(1, D), be1.reshape(1, D),
                W2, b2.reshape(1, D), g2.reshape(1, D), be2.reshape(1, D),
                W3, b3.reshape(1, 1))
